# baseline (speedup 1.0000x reference)
import jax
import jax.numpy as jnp
from jax import lax
from jax.experimental import pallas as pl
from jax.experimental.pallas import tpu as pltpu
from jax.experimental.pallas import tpu_sc as plsc

VOCAB = 1000000
D = 64
B = 16384 * 50

NC = 2
NS = 16
NW = NC * NS
B_PER_W = B // NW
IDX_ROW = 128
N_ROWS = B_PER_W // IDX_ROW
GB = 4
TB = 2


def _row_coords(r_global):
    t = lax.shift_right_logical(r_global, 7)
    tj = lax.bitwise_and(r_global, 127)
    return t, tj


def _embed_body(x_hbm, table_hbm, out_hbm, idx_v,
                g0, g1, g2, g3, t0, t1, i0, i1, i2, i3,
                gs0, gs1, gs2, gs3, os0, os1):
    wid = lax.axis_index("s") * NC + lax.axis_index("c")
    pltpu.sync_copy(x_hbm.at[wid], idx_v)
    row_base = wid * N_ROWS
    gbufs = (g0, g1, g2, g3)
    tbufs = (t0, t1)
    ibufs = (i0, i1, i2, i3)
    gsems = (gs0, gs1, gs2, gs3)
    osems = (os0, os1)
    iota16 = lax.iota(jnp.int32, 16)

    def fire_g(r, slot):
        ib = ibufs[slot]
        for c in range(8):
            ib[pl.ds(c * 16, 16)] = lax.shift_right_logical(
                idx_v[r, pl.ds(c * 16, 16)], 1)
        pltpu.async_copy(table_hbm.at[ib], gbufs[slot], gsems[slot])

    def drain_g(slot):
        pltpu.make_async_copy(
            table_hbm.at[ibufs[slot]], gbufs[slot], gsems[slot]).wait()

    def fire_outs(r, slot):
        t, tj = _row_coords(row_base + r)
        ob = t * 1024 + tj
        for ti in range(8):
            pltpu.async_copy(
                tbufs[slot].at[pl.ds(ti * 8, 8)],
                out_hbm.at[ob + ti * 128], osems[slot])

    def wait_outs(r, slot):
        t, tj = _row_coords(row_base + r)
        ob = t * 1024 + tj
        for ti in range(8):
            pltpu.make_async_copy(
                tbufs[slot].at[pl.ds(ti * 8, 8)],
                out_hbm.at[ob + ti * 128], osems[slot]).wait()

    djs = [jnp.bitwise_and(iota16 + j, 15) for j in range(16)]

    def transpose(r, gslot, tslot):
        gb, tb = gbufs[gslot], tbufs[tslot]

        @plsc.parallel_loop(0, IDX_ROW, step=16)
        def _bp(bp0):
            bpv = iota16 + bp0
            hv = jnp.bitwise_and(idx_v[r, pl.ds(bp0, 16)], 1) * 64
            for d0 in (0, 16, 32, 48):
                hd = hv + d0
                for j in range(16):
                    vals = plsc.load_gather(gb, [bpv, djs[j] + hd])
                    plsc.store_scatter(tb, [djs[j] + d0, bpv], vals)

    for r in range(GB):
        fire_g(r, r)

    @pl.loop(0, N_ROWS, step=GB)
    def _rows(r0):
        for j in range(GB):
            r = r0 + j
            gslot, tslot = j % GB, j % TB
            drain_g(gslot)
            if j < TB:
                pl.when(r0 > 0)(lambda rr=r, ts=tslot: wait_outs(rr - TB, ts))
            else:
                wait_outs(r - TB, tslot)
            transpose(r, gslot, tslot)
            fire_outs(r, tslot)
            pl.when(r0 < N_ROWS - GB)(
                lambda rr=r, gs=gslot: fire_g(rr + GB, gs))

    wait_outs(N_ROWS - 2, 0)
    wait_outs(N_ROWS - 1, 1)


@jax.jit
def _embed(x_flat3, table2):
    mesh = plsc.VectorSubcoreMesh(core_axis_name="c", subcore_axis_name="s")
    return pl.kernel(
        _embed_body,
        out_type=jax.ShapeDtypeStruct((51200, 8, 128), jnp.float32),
        mesh=mesh,
        compiler_params=pltpu.CompilerParams(
            use_tc_tiling_on_sc=True, needs_layout_passes=False),
        scratch_types=[
            pltpu.VMEM((N_ROWS, IDX_ROW), jnp.int32),
            pltpu.VMEM((IDX_ROW, 128), jnp.float32),
            pltpu.VMEM((IDX_ROW, 128), jnp.float32),
            pltpu.VMEM((IDX_ROW, 128), jnp.float32),
            pltpu.VMEM((IDX_ROW, 128), jnp.float32),
            pltpu.VMEM((D, 128), jnp.float32),
            pltpu.VMEM((D, 128), jnp.float32),
            pltpu.VMEM((IDX_ROW,), jnp.int32),
            pltpu.VMEM((IDX_ROW,), jnp.int32),
            pltpu.VMEM((IDX_ROW,), jnp.int32),
            pltpu.VMEM((IDX_ROW,), jnp.int32),
            pltpu.SemaphoreType.DMA,
            pltpu.SemaphoreType.DMA,
            pltpu.SemaphoreType.DMA,
            pltpu.SemaphoreType.DMA,
            pltpu.SemaphoreType.DMA,
            pltpu.SemaphoreType.DMA,
        ],
    )(x_flat3, table2)


def kernel(x, table):
    x_flat3 = x.T.reshape(NW, N_ROWS, IDX_ROW).astype(jnp.int32)
    table2 = table.reshape(VOCAB // 2, 2 * D)
    out5 = _embed(x_flat3, table2).reshape(50, 8, 128, 8, 128)
    return out5.transpose(2, 4, 0, 1, 3).reshape(16384, 50, D)


# speedup vs baseline: 1.2545x; 1.2545x over previous
import jax
import jax.numpy as jnp
from jax import lax
from jax.experimental import pallas as pl
from jax.experimental.pallas import tpu as pltpu
from jax.experimental.pallas import tpu_sc as plsc

VOCAB = 1000000
D = 64
B = 16384 * 50

NC = 2
NS = 16
NW = NC * NS
B_PER_W = B // NW
IDX_ROW = 128
N_ROWS = B_PER_W // IDX_ROW
GB = 4
TB = 2


def _row_coords(r_global):
    t = lax.shift_right_logical(r_global, 7)
    tj = lax.bitwise_and(r_global, 127)
    return t, tj


def _embed_body(x_hbm, table_hbm, out_hbm, idx_v,
                g0, g1, g2, g3, t0, t1,
                gs0, gs1, gs2, gs3, os0, os1):
    wid = lax.axis_index("s") * NC + lax.axis_index("c")
    pltpu.sync_copy(x_hbm.at[wid], idx_v)
    row_base = wid * N_ROWS
    gbufs = (g0, g1, g2, g3)
    tbufs = (t0, t1)
    gsems = (gs0, gs1, gs2, gs3)
    osems = (os0, os1)
    iota16 = lax.iota(jnp.int32, 16)

    def fire_g(r, slot):
        pltpu.async_copy(table_hbm.at[idx_v.at[r]], gbufs[slot], gsems[slot])

    def drain_g(r, slot):
        pltpu.make_async_copy(
            table_hbm.at[idx_v.at[r]], gbufs[slot], gsems[slot]).wait()

    def fire_outs(r, slot):
        t, tj = _row_coords(row_base + r)
        for ti in range(8):
            pltpu.async_copy(
                tbufs[slot].at[pl.ds(ti * 1024, 1024)],
                out_hbm.at[t, ti, tj], osems[slot])

    def wait_outs(r, slot):
        t, tj = _row_coords(row_base + r)
        for ti in range(8):
            pltpu.make_async_copy(
                tbufs[slot].at[pl.ds(ti * 1024, 1024)],
                out_hbm.at[t, ti, tj], osems[slot]).wait()

    djs = [jnp.bitwise_and(iota16 + j, 15) for j in range(16)]
    gdst = [djs[j] * 128 + iota16 for j in range(16)]

    def transpose(gslot, tslot):
        gb, tb = gbufs[gslot], tbufs[tslot]

        @plsc.parallel_loop(0, IDX_ROW, step=16, unroll=2)
        def _bp(bp0):
            bpv = iota16 + bp0
            for d0 in (0, 16, 32, 48):
                for j in range(16):
                    vals = plsc.load_gather(gb, [bpv, djs[j] + d0])
                    plsc.store_scatter(tb, [gdst[j] + (d0 * 128 + bp0)], vals)

    for r in range(GB):
        fire_g(r, r)

    @pl.loop(0, N_ROWS, step=GB)
    def _rows(r0):
        for j in range(GB):
            r = r0 + j
            gslot, tslot = j % GB, j % TB
            drain_g(r, gslot)
            if j < TB:
                pl.when(r0 > 0)(lambda rr=r, ts=tslot: wait_outs(rr - TB, ts))
            else:
                wait_outs(r - TB, tslot)
            transpose(gslot, tslot)
            fire_outs(r, tslot)
            pl.when(r0 < N_ROWS - GB)(
                lambda rr=r, gs=gslot: fire_g(rr + GB, gs))

    wait_outs(N_ROWS - 2, 0)
    wait_outs(N_ROWS - 1, 1)


@jax.jit
def _embed(x_flat3, table):
    mesh = plsc.VectorSubcoreMesh(core_axis_name="c", subcore_axis_name="s")
    return pl.kernel(
        _embed_body,
        out_type=jax.ShapeDtypeStruct((50, 8, 128, 1024), jnp.float32),
        mesh=mesh,
        compiler_params=pltpu.CompilerParams(
            use_tc_tiling_on_sc=False, needs_layout_passes=False),
        scratch_types=[
            pltpu.VMEM((N_ROWS, IDX_ROW), jnp.int32),
            pltpu.VMEM((IDX_ROW, D), jnp.float32),
            pltpu.VMEM((IDX_ROW, D), jnp.float32),
            pltpu.VMEM((IDX_ROW, D), jnp.float32),
            pltpu.VMEM((IDX_ROW, D), jnp.float32),
            pltpu.VMEM((8192,), jnp.float32),
            pltpu.VMEM((8192,), jnp.float32),
            pltpu.SemaphoreType.DMA,
            pltpu.SemaphoreType.DMA,
            pltpu.SemaphoreType.DMA,
            pltpu.SemaphoreType.DMA,
            pltpu.SemaphoreType.DMA,
            pltpu.SemaphoreType.DMA,
        ],
    )(x_flat3, table)


def kernel(x, table):
    x_flat3 = x.T.reshape(NW, N_ROWS, IDX_ROW).astype(jnp.int32)
    out5 = _embed(x_flat3, table).reshape(50, 8, 128, 8, 128)
    return out5.transpose(2, 4, 0, 1, 3).reshape(16384, 50, D)


# speedup vs baseline: 1.4009x; 1.1167x over previous
import jax
import jax.numpy as jnp
from jax import lax
from jax.experimental import pallas as pl
from jax.experimental.pallas import tpu as pltpu
from jax.experimental.pallas import tpu_sc as plsc

VOCAB = 1000000
D = 64
B = 16384 * 50

NC = 2
NS = 16
NW = NC * NS
B_PER_W = B // NW
IDX_ROW = 128
N_ROWS = B_PER_W // IDX_ROW
GB = 4
TB = 2


def _row_coords(r_global):
    t = lax.shift_right_logical(r_global, 7)
    tj = lax.bitwise_and(r_global, 127)
    return t, tj


def _embed_body(x_hbm, table_hbm, out_hbm, idx_v,
                g0, g1, g2, g3, t0, t1,
                gs0, gs1, gs2, gs3, os0, os1):
    wid = lax.axis_index("s") * NC + lax.axis_index("c")
    pltpu.sync_copy(x_hbm.at[wid], idx_v)
    row_base = wid * N_ROWS
    gbufs = (g0, g1, g2, g3)
    tbufs = (t0, t1)
    gsems = (gs0, gs1, gs2, gs3)
    osems = (os0, os1)
    iota16 = lax.iota(jnp.int32, 16)

    def fire_g(r, slot):
        pltpu.async_copy(table_hbm.at[idx_v.at[r]], gbufs[slot], gsems[slot])

    def drain_g(r, slot):
        pltpu.make_async_copy(
            table_hbm.at[idx_v.at[r]], gbufs[slot], gsems[slot]).wait()

    def fire_outs(r, slot):
        t, tj = _row_coords(row_base + r)
        for ti in range(8):
            pltpu.async_copy(
                tbufs[slot].at[pl.ds(ti * 1024, 1024)],
                out_hbm.at[t, ti, tj], osems[slot])

    def wait_outs(r, slot):
        t, tj = _row_coords(row_base + r)
        for ti in range(8):
            pltpu.make_async_copy(
                tbufs[slot].at[pl.ds(ti * 1024, 1024)],
                out_hbm.at[t, ti, tj], osems[slot]).wait()

    djs = [jnp.bitwise_and(iota16 + j, 15) for j in range(16)]
    gdst = [djs[j] * 128 + iota16 for j in range(16)]

    def transpose(gslot, tslot):
        gb, tb = gbufs[gslot], tbufs[tslot]

        @plsc.parallel_loop(0, IDX_ROW, step=16, unroll=4)
        def _bp(bp0):
            bpv = iota16 + bp0
            for d0 in (0, 16, 32, 48):
                for j in range(16):
                    vals = plsc.load_gather(gb, [bpv, djs[j] + d0])
                    plsc.store_scatter(tb, [gdst[j] + (d0 * 128 + bp0)], vals)

    for r in range(GB):
        fire_g(r, r)

    @pl.loop(0, N_ROWS, step=GB)
    def _rows(r0):
        for j in range(GB):
            r = r0 + j
            gslot, tslot = j % GB, j % TB
            drain_g(r, gslot)
            if j < TB:
                pl.when(r0 > 0)(lambda rr=r, ts=tslot: wait_outs(rr - TB, ts))
            else:
                wait_outs(r - TB, tslot)
            transpose(gslot, tslot)
            fire_outs(r, tslot)
            pl.when(r0 < N_ROWS - GB)(
                lambda rr=r, gs=gslot: fire_g(rr + GB, gs))

    wait_outs(N_ROWS - 2, 0)
    wait_outs(N_ROWS - 1, 1)


@jax.jit
def _embed(x_flat3, table):
    mesh = plsc.VectorSubcoreMesh(core_axis_name="c", subcore_axis_name="s")
    return pl.kernel(
        _embed_body,
        out_type=jax.ShapeDtypeStruct((50, 8, 128, 1024), jnp.float32),
        mesh=mesh,
        compiler_params=pltpu.CompilerParams(
            use_tc_tiling_on_sc=False, needs_layout_passes=False),
        scratch_types=[
            pltpu.VMEM((N_ROWS, IDX_ROW), jnp.int32),
            pltpu.VMEM((IDX_ROW, D), jnp.float32),
            pltpu.VMEM((IDX_ROW, D), jnp.float32),
            pltpu.VMEM((IDX_ROW, D), jnp.float32),
            pltpu.VMEM((IDX_ROW, D), jnp.float32),
            pltpu.VMEM((8192,), jnp.float32),
            pltpu.VMEM((8192,), jnp.float32),
            pltpu.SemaphoreType.DMA,
            pltpu.SemaphoreType.DMA,
            pltpu.SemaphoreType.DMA,
            pltpu.SemaphoreType.DMA,
            pltpu.SemaphoreType.DMA,
            pltpu.SemaphoreType.DMA,
        ],
    )(x_flat3, table)


def kernel(x, table):
    x_flat3 = x.T.reshape(NW, N_ROWS, IDX_ROW).astype(jnp.int32)
    out5 = _embed(x_flat3, table).reshape(50, 8, 128, 8, 128)
    return out5.transpose(2, 4, 0, 1, 3).reshape(16384, 50, D)


# speedup vs baseline: 2.4843x; 1.7733x over previous
import jax
import jax.numpy as jnp
from jax import lax
from jax.experimental import pallas as pl
from jax.experimental.pallas import tpu as pltpu
from jax.experimental.pallas import tpu_sc as plsc

VOCAB = 1000000
D = 64
B = 16384 * 50

NC = 2
NS = 16
NW = NC * NS
B_PER_W = B // NW
IDX_ROW = 128
N_ROWS = B_PER_W // IDX_ROW
GB = 4
TB = 2


def _row_coords(r_global):
    t = lax.shift_right_logical(r_global, 7)
    tj = lax.bitwise_and(r_global, 127)
    return t, tj


def _embed_body(x_hbm, table_hbm, out_hbm, idx_v,
                g0, g1, g2, g3, t0, t1,
                gs0, gs1, gs2, gs3, os0, os1):
    wid = lax.axis_index("s") * NC + lax.axis_index("c")
    pltpu.sync_copy(x_hbm.at[wid], idx_v)
    row_base = wid * N_ROWS
    gbufs = (g0, g1, g2, g3)
    tbufs = (t0, t1)
    gsems = (gs0, gs1, gs2, gs3)
    osems = (os0, os1)
    iota16 = lax.iota(jnp.int32, 16)

    def fire_g(r, slot):
        pltpu.async_copy(table_hbm.at[idx_v.at[r]], gbufs[slot], gsems[slot])

    def drain_g(r, slot):
        pltpu.make_async_copy(
            table_hbm.at[idx_v.at[r]], gbufs[slot], gsems[slot]).wait()

    def fire_outs(r, slot):
        t, tj = _row_coords(row_base + r)
        for ti in range(8):
            pltpu.async_copy(
                tbufs[slot].at[pl.ds(ti * 1024, 1024)],
                out_hbm.at[t, ti, tj], osems[slot])

    def wait_outs(r, slot):
        t, tj = _row_coords(row_base + r)
        for ti in range(8):
            pltpu.make_async_copy(
                tbufs[slot].at[pl.ds(ti * 1024, 1024)],
                out_hbm.at[t, ti, tj], osems[slot]).wait()

    djs = [jnp.bitwise_and(iota16 + j, 15) for j in range(16)]
    gdst = [djs[j] * 128 + iota16 for j in range(16)]

    def transpose(gslot, tslot):
        gb, tb = gbufs[gslot], tbufs[tslot]

        @plsc.parallel_loop(0, IDX_ROW, step=16, unroll=4)
        def _bp(bp0):
            bpv = iota16 + bp0
            for d0 in (0, 16, 32, 48):
                for j in range(16):
                    vals = plsc.load_gather(gb, [bpv, djs[j] + d0])
                    plsc.store_scatter(tb, [gdst[j] + (d0 * 128 + bp0)], vals)

    for r in range(GB):
        fire_g(r, r)

    @pl.loop(0, N_ROWS, step=GB)
    def _rows(r0):
        for j in range(GB):
            r = r0 + j
            gslot, tslot = j % GB, j % TB
            drain_g(r, gslot)
            if j < TB:
                pl.when(r0 > 0)(lambda rr=r, ts=tslot: wait_outs(rr - TB, ts))
            else:
                wait_outs(r - TB, tslot)
            transpose(gslot, tslot)
            fire_outs(r, tslot)
            pl.when(r0 < N_ROWS - GB)(
                lambda rr=r, gs=gslot: fire_g(rr + GB, gs))

    wait_outs(N_ROWS - 2, 0)
    wait_outs(N_ROWS - 1, 1)


N_TCOL = 7812
TCOL_W = N_TCOL // NW


def _ttrans_body(tt_hbm, tail_hbm, out_hbm, ga, gb_, ta, tb_,
                 isa, isb, osa, osb):
    wid = lax.axis_index("s") * NC + lax.axis_index("c")
    iota16 = lax.iota(jnp.int32, 16)
    djs = [jnp.bitwise_and(iota16 + j, 15) for j in range(16)]
    gbufs, tbufs = (ga, gb_), (ta, tb_)
    isems, osems = (isa, isb), (osa, osb)
    ncols = TCOL_W + jnp.where(wid == NW - 1, 4, 0)
    col0 = wid * TCOL_W

    def fire_in(col, slot):
        for ti in range(8):
            pltpu.async_copy(
                tt_hbm.at[pl.ds(ti * 8, 8), pl.ds(col * 128, 128)],
                gbufs[slot].at[pl.ds(ti * 8, 8)], isems[slot])

    def drain_in(col, slot):
        for ti in range(8):
            pltpu.make_async_copy(
                tt_hbm.at[pl.ds(ti * 8, 8), pl.ds(col * 128, 128)],
                gbufs[slot].at[pl.ds(ti * 8, 8)], isems[slot]).wait()

    def fire_out(col, slot):
        pltpu.async_copy(
            tbufs[slot], out_hbm.at[pl.ds(col * 64, 64)], osems[slot])

    def wait_out(col, slot):
        pltpu.make_async_copy(
            tbufs[slot], out_hbm.at[pl.ds(col * 64, 64)], osems[slot]).wait()

    def transpose(gslot, tslot):
        g, t = gbufs[gslot], tbufs[tslot]

        @plsc.parallel_loop(0, 128, step=16, unroll=4)
        def _c(c0):
            cvec = iota16 + c0
            rvec = lax.shift_right_logical(cvec, 1)
            hvec = jnp.bitwise_and(cvec, 1) * 64
            for d0 in (0, 16, 32, 48):
                for j in range(16):
                    vals = plsc.load_gather(g, [djs[j] + d0, cvec])
                    plsc.store_scatter(
                        t, [rvec, hvec + (djs[j] + d0)], vals)

    fire_in(col0, 0)
    fire_in(col0 + 1, 1)

    @pl.loop(0, ncols, step=2)
    def _cols(c):
        for par in range(2):
            col = col0 + c + par
            drain_in(col, par)
            pl.when(c > 0)(lambda cc=col, s=par: wait_out(cc - 2, s))
            transpose(par, par)
            fire_out(col, par)
            pl.when(c + par + 2 < ncols)(
                lambda cc=col, s=par: fire_in(cc + 2, s))

    wait_out(col0 + ncols - 2, 0)
    wait_out(col0 + ncols - 1, 1)

    @pl.when(wid == NW - 1)
    def _tail():
        pltpu.sync_copy(tail_hbm, ta.at[pl.ds(0, 32)])
        pltpu.sync_copy(ta.at[pl.ds(0, 32)],
                        out_hbm.at[pl.ds(N_TCOL * 64, 32)])


@jax.jit
def _ttrans(table_t, tail2):
    mesh = plsc.VectorSubcoreMesh(core_axis_name="c", subcore_axis_name="s")
    return pl.kernel(
        _ttrans_body,
        out_type=jax.ShapeDtypeStruct((VOCAB // 2, 128), jnp.float32),
        mesh=mesh,
        compiler_params=pltpu.CompilerParams(
            use_tc_tiling_on_sc=True, needs_layout_passes=False),
        scratch_types=[
            pltpu.VMEM((D, 128), jnp.float32),
            pltpu.VMEM((D, 128), jnp.float32),
            pltpu.VMEM((D, 128), jnp.float32),
            pltpu.VMEM((D, 128), jnp.float32),
            pltpu.SemaphoreType.DMA,
            pltpu.SemaphoreType.DMA,
            pltpu.SemaphoreType.DMA,
            pltpu.SemaphoreType.DMA,
        ],
    )(table_t, tail2)


@jax.jit
def _embed(x_flat3, table):
    mesh = plsc.VectorSubcoreMesh(core_axis_name="c", subcore_axis_name="s")
    return pl.kernel(
        _embed_body,
        out_type=jax.ShapeDtypeStruct((50, 8, 128, 1024), jnp.float32),
        mesh=mesh,
        compiler_params=pltpu.CompilerParams(
            use_tc_tiling_on_sc=False, needs_layout_passes=False),
        scratch_types=[
            pltpu.VMEM((N_ROWS, IDX_ROW), jnp.int32),
            pltpu.VMEM((IDX_ROW, D), jnp.float32),
            pltpu.VMEM((IDX_ROW, D), jnp.float32),
            pltpu.VMEM((IDX_ROW, D), jnp.float32),
            pltpu.VMEM((IDX_ROW, D), jnp.float32),
            pltpu.VMEM((8192,), jnp.float32),
            pltpu.VMEM((8192,), jnp.float32),
            pltpu.SemaphoreType.DMA,
            pltpu.SemaphoreType.DMA,
            pltpu.SemaphoreType.DMA,
            pltpu.SemaphoreType.DMA,
            pltpu.SemaphoreType.DMA,
            pltpu.SemaphoreType.DMA,
        ],
    )(x_flat3, table)


def kernel(x, table):
    x_flat3 = x.T.reshape(NW, N_ROWS, IDX_ROW).astype(jnp.int32)
    tail2 = table[N_TCOL * 128:].reshape(32, 128)
    table_rm = _ttrans(table.T, tail2).reshape(VOCAB, D)
    out5 = _embed(x_flat3, table_rm).reshape(50, 8, 128, 8, 128)
    return out5.transpose(2, 4, 0, 1, 3).reshape(16384, 50, D)
